# baseline (speedup 1.0000x reference)
import functools

import jax
import jax.numpy as jnp
from jax.experimental import pallas as pl
from jax.experimental.pallas import tpu as pltpu

H = 1024
I = 4096
E = 8
TOK = 4096
K = 2

B = 512
IB = 2048
NB = (TOK * K) // B + E
NI = I // IB


def _gelu_tanh(x):
    c = jnp.float32(0.7978845608028654)
    return 0.5 * x * (1.0 + jnp.tanh(c * (x + 0.044715 * (x * x * x))))


def _sweep_body_first(meta_ref, gx_ref, w_ref, wg_ref, wu_ref, wd_ref, out_ref):
    b = pl.program_id(0)
    nblocks = meta_ref[NB]

    @pl.when(b < nblocks)
    def _():
        xb = gx_ref[...]
        wg = wg_ref[0].astype(jnp.bfloat16)
        wu = wu_ref[0].astype(jnp.bfloat16)
        wd = wd_ref[0].astype(jnp.bfloat16)
        g = jnp.dot(xb, wg, preferred_element_type=jnp.float32)
        u = jnp.dot(xb, wu, preferred_element_type=jnp.float32)
        h = (_gelu_tanh(g) * u * w_ref[...]).astype(jnp.bfloat16)
        out_ref[...] = jnp.dot(h, wd, preferred_element_type=jnp.float32)


def _sweep_body_acc(meta_ref, gx_ref, w_ref, wg_ref, wu_ref, wd_ref, acc_ref,
                    out_ref):
    b = pl.program_id(0)
    nblocks = meta_ref[NB]

    @pl.when(b < nblocks)
    def _():
        xb = gx_ref[...]
        wg = wg_ref[0].astype(jnp.bfloat16)
        wu = wu_ref[0].astype(jnp.bfloat16)
        wd = wd_ref[0].astype(jnp.bfloat16)
        g = jnp.dot(xb, wg, preferred_element_type=jnp.float32)
        u = jnp.dot(xb, wu, preferred_element_type=jnp.float32)
        h = (_gelu_tanh(g) * u * w_ref[...]).astype(jnp.bfloat16)
        out_ref[...] = acc_ref[...] + jnp.dot(
            h, wd, preferred_element_type=jnp.float32)


def _sweep(j, meta, gx, w_pad, W_gate, W_up, W_down, acc):
    row_spec = pl.BlockSpec((B, H), lambda b, m: (b, 0))
    in_specs = [
        row_spec,
        pl.BlockSpec((B, 1), lambda b, m: (b, 0)),
        pl.BlockSpec((1, H, IB), lambda b, m: (m[b], 0, j)),
        pl.BlockSpec((1, H, IB), lambda b, m: (m[b], 0, j)),
        pl.BlockSpec((1, IB, H), lambda b, m: (m[b], j, 0)),
    ]
    args = [meta, gx, w_pad, W_gate, W_up, W_down]
    body = _sweep_body_first
    if acc is not None:
        in_specs.append(row_spec)
        args.append(acc)
        body = _sweep_body_acc
    grid_spec = pltpu.PrefetchScalarGridSpec(
        num_scalar_prefetch=1,
        grid=(NB,),
        in_specs=in_specs,
        out_specs=row_spec,
    )
    return pl.pallas_call(
        body,
        grid_spec=grid_spec,
        out_shape=jax.ShapeDtypeStruct((NB * B, H), jnp.float32),
        compiler_params=pltpu.CompilerParams(
            dimension_semantics=("arbitrary",),
            vmem_limit_bytes=100 * 1024 * 1024,
        ),
    )(*args)


def _grouped_mlp(meta, gx, w_pad, W_gate, W_up, W_down):
    ko = _sweep(0, meta, gx, w_pad, W_gate, W_up, W_down, None)
    for j in range(1, NI):
        ko = _sweep(j, meta, gx, w_pad, W_gate, W_up, W_down, ko)
    return ko


def kernel(x, selected_experts, routing_weights, W_gate, W_up, W_down):
    e = selected_experts.reshape(-1).astype(jnp.int32)
    oh = (e[:, None] == jnp.arange(E, dtype=jnp.int32)[None, :]).astype(jnp.int32)
    csum = jnp.cumsum(oh, axis=0)
    counts = csum[-1]
    within = jnp.take_along_axis(csum, e[:, None], axis=1)[:, 0] - 1

    nb = (counts + (B - 1)) // B
    first_block = jnp.concatenate([jnp.zeros((1,), jnp.int32),
                                   jnp.cumsum(nb).astype(jnp.int32)])
    nblocks = first_block[E]
    pos = first_block[e] * B + within

    blk = jnp.arange(NB, dtype=jnp.int32)
    blk_c = jnp.minimum(blk, nblocks - 1)
    block_expert = jnp.searchsorted(first_block[1:], blk_c, side="right").astype(jnp.int32)
    meta = jnp.concatenate([block_expert, nblocks[None]])

    pair_of_row = jnp.zeros((NB * B,), jnp.int32).at[pos].set(
        jnp.arange(TOK * K, dtype=jnp.int32), mode="drop")
    gx = jnp.take(x.astype(jnp.bfloat16), pair_of_row // K, axis=0)

    rw = routing_weights.astype(jnp.float32)
    w_pad = jnp.zeros((NB * B, 1), jnp.float32).at[pos, 0].set(
        rw.reshape(-1), mode="drop")

    ko = _grouped_mlp(meta, gx, w_pad, W_gate, W_up, W_down)

    pos2 = pos.reshape(TOK, K)
    out = jnp.take(ko, pos2[:, 0], axis=0) + jnp.take(ko, pos2[:, 1], axis=0)
    return out


# speedup vs baseline: 1.0640x; 1.0640x over previous
import functools

import jax
import jax.numpy as jnp
from jax.experimental import pallas as pl
from jax.experimental.pallas import tpu as pltpu

H = 1024
I = 4096
E = 8
TOK = 4096
K = 2

B = 512
IB = 2048
NB = (TOK * K) // B + E
NI = I // IB


def _gelu_tanh(x):
    c = jnp.float32(0.7978845608028654)
    return 0.5 * x * (1.0 + jnp.tanh(c * (x + 0.044715 * (x * x * x))))


def _sweep_body_first(meta_ref, gx_ref, wg_ref, wu_ref, wd_ref, out_ref):
    b = pl.program_id(0)
    nblocks = meta_ref[NB]

    @pl.when(b < nblocks)
    def _():
        xb = gx_ref[...]
        wg = wg_ref[0].astype(jnp.bfloat16)
        wu = wu_ref[0].astype(jnp.bfloat16)
        wd = wd_ref[0].astype(jnp.bfloat16)
        g = jnp.dot(xb, wg, preferred_element_type=jnp.float32)
        u = jnp.dot(xb, wu, preferred_element_type=jnp.float32)
        h = (_gelu_tanh(g) * u).astype(jnp.bfloat16)
        out_ref[...] = jnp.dot(h, wd, preferred_element_type=jnp.float32)


def _sweep_body_acc(meta_ref, gx_ref, wg_ref, wu_ref, wd_ref, acc_ref, out_ref):
    b = pl.program_id(0)
    nblocks = meta_ref[NB]

    @pl.when(b < nblocks)
    def _():
        xb = gx_ref[...]
        wg = wg_ref[0].astype(jnp.bfloat16)
        wu = wu_ref[0].astype(jnp.bfloat16)
        wd = wd_ref[0].astype(jnp.bfloat16)
        g = jnp.dot(xb, wg, preferred_element_type=jnp.float32)
        u = jnp.dot(xb, wu, preferred_element_type=jnp.float32)
        h = (_gelu_tanh(g) * u).astype(jnp.bfloat16)
        out_ref[...] = acc_ref[...] + jnp.dot(
            h, wd, preferred_element_type=jnp.float32)


def _sweep(j, meta, gx, W_gate, W_up, W_down, acc):
    row_spec = pl.BlockSpec((B, H), lambda b, m: (b, 0))
    in_specs = [
        row_spec,
        pl.BlockSpec((1, H, IB), lambda b, m: (m[b], 0, j)),
        pl.BlockSpec((1, H, IB), lambda b, m: (m[b], 0, j)),
        pl.BlockSpec((1, IB, H), lambda b, m: (m[b], j, 0)),
    ]
    args = [meta, gx, W_gate, W_up, W_down]
    body = _sweep_body_first
    if acc is not None:
        in_specs.append(row_spec)
        args.append(acc)
        body = _sweep_body_acc
    grid_spec = pltpu.PrefetchScalarGridSpec(
        num_scalar_prefetch=1,
        grid=(NB,),
        in_specs=in_specs,
        out_specs=row_spec,
    )
    return pl.pallas_call(
        body,
        grid_spec=grid_spec,
        out_shape=jax.ShapeDtypeStruct((NB * B, H), jnp.float32),
        compiler_params=pltpu.CompilerParams(
            dimension_semantics=("parallel",),
            vmem_limit_bytes=100 * 1024 * 1024,
        ),
    )(*args)


def _grouped_mlp(meta, gx, W_gate, W_up, W_down):
    ko = _sweep(0, meta, gx, W_gate, W_up, W_down, None)
    for j in range(1, NI):
        ko = _sweep(j, meta, gx, W_gate, W_up, W_down, ko)
    return ko


def kernel(x, selected_experts, routing_weights, W_gate, W_up, W_down):
    e = selected_experts.reshape(-1).astype(jnp.int32)
    oh = (e[:, None] == jnp.arange(E, dtype=jnp.int32)[None, :]).astype(jnp.int32)
    csum = jnp.cumsum(oh, axis=0)
    counts = csum[-1]
    within = jnp.take_along_axis(csum, e[:, None], axis=1)[:, 0] - 1

    nb = (counts + (B - 1)) // B
    first_block = jnp.concatenate([jnp.zeros((1,), jnp.int32),
                                   jnp.cumsum(nb).astype(jnp.int32)])
    nblocks = first_block[E]
    pos = first_block[e] * B + within

    blk = jnp.arange(NB, dtype=jnp.int32)
    blk_c = jnp.minimum(blk, nblocks - 1)
    block_expert = jnp.searchsorted(first_block[1:], blk_c, side="right").astype(jnp.int32)
    meta = jnp.concatenate([block_expert, nblocks[None]])

    pair_of_row = jnp.zeros((NB * B,), jnp.int32).at[pos].set(
        jnp.arange(TOK * K, dtype=jnp.int32), mode="drop")
    gx = jnp.take(x.astype(jnp.bfloat16), pair_of_row // K, axis=0)

    ko = _grouped_mlp(meta, gx, W_gate, W_up, W_down)

    rw = routing_weights.astype(jnp.float32)
    pos2 = pos.reshape(TOK, K)
    out = (jnp.take(ko, pos2[:, 0], axis=0) * rw[:, 0:1]
           + jnp.take(ko, pos2[:, 1], axis=0) * rw[:, 1:2])
    return out
